# baseline (speedup 1.0000x reference)
import jax
import jax.numpy as jnp
from jax.experimental import pallas as pl
from jax.experimental.pallas import tpu as pltpu

HIDDEN = 1024
EXPERT_INTER = 1024
NUM_EXPERTS = 16
VOCAB = 100000
N_TOKENS = 128
NBUF = 2


def _moe_body(tid_ref, x_ref, gu_hbm, dn_hbm, out_ref,
              gub, dnb, gsem, dsem):
    def gu_copy(e, slot):
        return pltpu.make_async_copy(gu_hbm.at[e], gub.at[slot], gsem.at[slot])

    def dn_copy(e, slot):
        return pltpu.make_async_copy(dn_hbm.at[e], dnb.at[slot], dsem.at[slot])

    def start_expert(e, slot):
        gu_copy(e, slot).start()
        dn_copy(e, slot).start()

    for p in range(NBUF):
        start_expert(p, p)

    tid = jnp.clip(tid_ref[...], 0, VOCAB - 1)
    eid = jax.lax.rem(tid, NUM_EXPERTS)

    acc = jnp.zeros((N_TOKENS, HIDDEN), jnp.float32)
    for e in range(NUM_EXPERTS):
        slot = e % NBUF
        mask = eid == e
        x = jnp.where(mask, x_ref[...], 0.0).astype(jnp.bfloat16)
        gu_copy(e, slot).wait()
        gate = jnp.dot(x, gub[slot, :, 0:EXPERT_INTER].astype(jnp.bfloat16),
                       preferred_element_type=jnp.float32)
        up = jnp.dot(x, gub[slot, :, EXPERT_INTER:].astype(jnp.bfloat16),
                     preferred_element_type=jnp.float32)
        act = (gate * jax.nn.sigmoid(gate) * up).astype(jnp.bfloat16)
        dn_copy(e, slot).wait()
        acc = acc + jnp.dot(act, dnb[slot].astype(jnp.bfloat16),
                            preferred_element_type=jnp.float32)
        if e + NBUF < NUM_EXPERTS:
            start_expert(e + NBUF, slot)
    out_ref[...] = acc


def kernel(x, token_ids, gate_up_proj, down_proj):
    n = x.shape[0]
    tid2d = token_ids.reshape(n, 1).astype(jnp.int32)
    return pl.pallas_call(
        _moe_body,
        in_specs=[
            pl.BlockSpec(memory_space=pltpu.MemorySpace.VMEM),
            pl.BlockSpec(memory_space=pltpu.MemorySpace.VMEM),
            pl.BlockSpec(memory_space=pltpu.MemorySpace.HBM),
            pl.BlockSpec(memory_space=pltpu.MemorySpace.HBM),
        ],
        out_specs=pl.BlockSpec(memory_space=pltpu.MemorySpace.VMEM),
        out_shape=jax.ShapeDtypeStruct((n, HIDDEN), jnp.float32),
        scratch_shapes=[
            pltpu.VMEM((NBUF, HIDDEN, 2 * EXPERT_INTER), jnp.float32),
            pltpu.VMEM((NBUF, EXPERT_INTER, HIDDEN), jnp.float32),
            pltpu.SemaphoreType.DMA((NBUF,)),
            pltpu.SemaphoreType.DMA((NBUF,)),
        ],
    )(tid2d, x, gate_up_proj, down_proj)


# speedup vs baseline: 1.0155x; 1.0155x over previous
import jax
import jax.numpy as jnp
from jax.experimental import pallas as pl
from jax.experimental.pallas import tpu as pltpu

HIDDEN = 1024
EXPERT_INTER = 1024
NUM_EXPERTS = 16
VOCAB = 100000
N_TOKENS = 128
NBUF = 2


def _moe_body(tid_ref, x_ref, gu_hbm, dn_hbm, out_ref,
              gateb, upb, dnb, gsem, usem, dsem):
    def gate_copy(e, slot):
        return pltpu.make_async_copy(
            gu_hbm.at[e, :, 0:EXPERT_INTER], gateb.at[slot], gsem.at[slot])

    def up_copy(e, slot):
        return pltpu.make_async_copy(
            gu_hbm.at[e, :, EXPERT_INTER:2 * EXPERT_INTER],
            upb.at[slot], usem.at[slot])

    def dn_copy(e, slot):
        return pltpu.make_async_copy(dn_hbm.at[e], dnb.at[slot], dsem.at[slot])

    def start_expert(e, slot):
        gate_copy(e, slot).start()
        up_copy(e, slot).start()
        dn_copy(e, slot).start()

    for p in range(NBUF):
        start_expert(p, p)

    tid = jnp.clip(tid_ref[...], 0, VOCAB - 1)
    eid = jax.lax.rem(tid, NUM_EXPERTS)

    acc = jnp.zeros((N_TOKENS, HIDDEN), jnp.float32)
    for e in range(NUM_EXPERTS):
        slot = e % NBUF
        mask = eid == e
        x = jnp.where(mask, x_ref[...], 0.0).astype(jnp.bfloat16)
        gate_copy(e, slot).wait()
        gate = jnp.dot(x, gateb[slot].astype(jnp.bfloat16),
                       preferred_element_type=jnp.float32)
        up_copy(e, slot).wait()
        up = jnp.dot(x, upb[slot].astype(jnp.bfloat16),
                     preferred_element_type=jnp.float32)
        act = (gate * jax.nn.sigmoid(gate) * up).astype(jnp.bfloat16)
        dn_copy(e, slot).wait()
        acc = acc + jnp.dot(act, dnb[slot].astype(jnp.bfloat16),
                            preferred_element_type=jnp.float32)
        if e + NBUF < NUM_EXPERTS:
            start_expert(e + NBUF, slot)
    out_ref[...] = acc


def kernel(x, token_ids, gate_up_proj, down_proj):
    n = x.shape[0]
    tid2d = token_ids.reshape(n, 1).astype(jnp.int32)
    return pl.pallas_call(
        _moe_body,
        in_specs=[
            pl.BlockSpec(memory_space=pltpu.MemorySpace.VMEM),
            pl.BlockSpec(memory_space=pltpu.MemorySpace.VMEM),
            pl.BlockSpec(memory_space=pltpu.MemorySpace.HBM),
            pl.BlockSpec(memory_space=pltpu.MemorySpace.HBM),
        ],
        out_specs=pl.BlockSpec(memory_space=pltpu.MemorySpace.VMEM),
        out_shape=jax.ShapeDtypeStruct((n, HIDDEN), jnp.float32),
        scratch_shapes=[
            pltpu.VMEM((NBUF, HIDDEN, EXPERT_INTER), jnp.float32),
            pltpu.VMEM((NBUF, HIDDEN, EXPERT_INTER), jnp.float32),
            pltpu.VMEM((NBUF, EXPERT_INTER, HIDDEN), jnp.float32),
            pltpu.SemaphoreType.DMA((NBUF,)),
            pltpu.SemaphoreType.DMA((NBUF,)),
            pltpu.SemaphoreType.DMA((NBUF,)),
        ],
    )(tid2d, x, gate_up_proj, down_proj)
